# baseline (speedup 1.0000x reference)
import functools

import jax
import jax.numpy as jnp
from jax import lax
from jax.experimental import pallas as pl
from jax.experimental.pallas import tpu as pltpu
from jax.experimental.pallas import tpu_sc as plsc

B = 16384
F = 32
NC = 2
NS = 16
L = 16
NW = NC * NS
BPW = B // NW


def _mf_dot_body(uid_hbm, iid_hbm, ut_hbm, it_hbm, out_hbm,
                 uidx_v, iidx_v, ucols_v, icols_v, out_v, sem):
  wid = lax.axis_index("s") * NC + lax.axis_index("c")
  base = wid * BPW

  pltpu.sync_copy(uid_hbm.at[pl.ds(base, BPW)], uidx_v)
  pltpu.sync_copy(iid_hbm.at[pl.ds(base, BPW)], iidx_v)

  copies = []
  for f in range(F):
    copies.append(pltpu.async_copy(ut_hbm.at[f].at[uidx_v], ucols_v.at[f], sem))
    copies.append(pltpu.async_copy(it_hbm.at[f].at[iidx_v], icols_v.at[f], sem))
  for c in copies:
    c.wait()

  def body(g, _):
    sl = pl.ds(g * L, L)
    acc = jnp.zeros((L,), jnp.float32)
    for f in range(F):
      acc = acc + ucols_v[f, sl] * icols_v[f, sl]
    out_v[sl] = acc
    return 0

  lax.fori_loop(0, BPW // L, body, 0)

  pltpu.sync_copy(out_v, out_hbm.at[pl.ds(base, BPW)])


_mf_dot = functools.partial(
    pl.kernel,
    out_type=jax.ShapeDtypeStruct((B,), jnp.float32),
    mesh=plsc.VectorSubcoreMesh(core_axis_name="c", subcore_axis_name="s"),
    scratch_types=[
        pltpu.VMEM((BPW,), jnp.int32),
        pltpu.VMEM((BPW,), jnp.int32),
        pltpu.VMEM((F, BPW), jnp.float32),
        pltpu.VMEM((F, BPW), jnp.float32),
        pltpu.VMEM((BPW,), jnp.float32),
        pltpu.SemaphoreType.DMA,
    ],
    compiler_params=pltpu.CompilerParams(
        needs_layout_passes=False, use_tc_tiling_on_sc=False),
)(_mf_dot_body)


@jax.jit
def kernel(user_ids, item_ids, user_table, item_table):
  return _mf_dot(user_ids.astype(jnp.int32), item_ids.astype(jnp.int32),
                 user_table.T, item_table.T)


# speedup vs baseline: 5.6124x; 5.6124x over previous
import functools

import jax
import jax.numpy as jnp
from jax import lax
from jax.experimental import pallas as pl
from jax.experimental.pallas import tpu as pltpu
from jax.experimental.pallas import tpu_sc as plsc

B = 16384
F = 32
RPL = 4
LINES = 250000
NC = 2
NS = 16
L = 16
NW = NC * NS
BPW = B // NW
HALF = BPW // 2
CH = 128


def _mf_dot_body(uid_hbm, iid_hbm, ut_hbm, it_hbm, out_hbm,
                 uidx_v, iidx_v, ulidx_v, ilidx_v,
                 urows_v, irows_v, out_v, sem):
  wid = lax.axis_index("s") * NC + lax.axis_index("c")
  base = wid * BPW

  pltpu.sync_copy(uid_hbm.at[pl.ds(base, BPW)], uidx_v)
  pltpu.sync_copy(iid_hbm.at[pl.ds(base, BPW)], iidx_v)

  def mkline(g, _):
    sl = pl.ds(g * L, L)
    ulidx_v[sl] = lax.shift_right_logical(uidx_v[sl], 2)
    ilidx_v[sl] = lax.shift_right_logical(iidx_v[sl], 2)
    return 0

  lax.fori_loop(0, BPW // L, mkline, 0)

  iota = lax.iota(jnp.int32, L)

  for h in range(BPW // HALF):
    copies = []
    for j in range(HALF // CH):
      isl = pl.ds(h * HALF + j * CH, CH)
      dsl = pl.ds(j * CH, CH)
      copies.append(pltpu.async_copy(
          ut_hbm.at[ulidx_v.at[isl]], urows_v.at[dsl], sem))
      copies.append(pltpu.async_copy(
          it_hbm.at[ilidx_v.at[isl]], irows_v.at[dsl], sem))
    for c in copies:
      c.wait()

    def body(g, _):
      sl = pl.ds(h * HALF + g * L, L)
      ucol = (uidx_v[sl] & 3) * F
      icol = (iidx_v[sl] & 3) * F
      row = g * L + iota
      acc = jnp.zeros((L,), jnp.float32)
      for f in range(F):
        u = plsc.load_gather(urows_v, [row, ucol + f])
        v = plsc.load_gather(irows_v, [row, icol + f])
        acc = acc + u * v
      out_v[sl] = acc
      return 0

    lax.fori_loop(0, HALF // L, body, 0)

  pltpu.sync_copy(out_v, out_hbm.at[pl.ds(base, BPW)])


_mf_dot = functools.partial(
    pl.kernel,
    out_type=jax.ShapeDtypeStruct((B,), jnp.float32),
    mesh=plsc.VectorSubcoreMesh(core_axis_name="c", subcore_axis_name="s"),
    scratch_types=[
        pltpu.VMEM((BPW,), jnp.int32),
        pltpu.VMEM((BPW,), jnp.int32),
        pltpu.VMEM((BPW,), jnp.int32),
        pltpu.VMEM((BPW,), jnp.int32),
        pltpu.VMEM((HALF, RPL * F), jnp.float32),
        pltpu.VMEM((HALF, RPL * F), jnp.float32),
        pltpu.VMEM((BPW,), jnp.float32),
        pltpu.SemaphoreType.DMA,
    ],
    compiler_params=pltpu.CompilerParams(
        needs_layout_passes=False, use_tc_tiling_on_sc=False),
)(_mf_dot_body)


@jax.jit
def kernel(user_ids, item_ids, user_table, item_table):
  return _mf_dot(user_ids.astype(jnp.int32), item_ids.astype(jnp.int32),
                 user_table.reshape(LINES, RPL * F),
                 item_table.reshape(LINES, RPL * F))
